# baseline (speedup 1.0000x reference)
import functools

import jax
import jax.numpy as jnp
from jax import lax
from jax.experimental import pallas as pl
from jax.experimental.pallas import tpu as pltpu
from jax.experimental.pallas import tpu_sc as plsc


B, N, M, C = 2, 4096, 4096, 3
SCW = 256
MT = M - SCW
NW = 32
RW = N // NW
CW = 16
NPH2 = SCW // CW
L = 16
UNROLL = 4
BN = 512


_GDN = lax.GatherDimensionNumbers(
    offset_dims=(), collapsed_slice_dims=(0,), start_index_map=(0,))


def _rot(v, s):
    idx = (lax.iota(jnp.int32, L) + s) % L
    return lax.gather(v, idx[:, None], _GDN, slice_sizes=(1,),
                      mode=lax.GatherScatterMode.PROMISE_IN_BOUNDS)


def _lane_min_all(v):
    for s in (8, 4, 2, 1):
        v = jnp.minimum(v, _rot(v, s))
    return v


def _phase(sx, sy, sz, bx, by, bz, outv, n_small, n_big):
    nchunks = n_big // (L * UNROLL)
    lane = lax.iota(jnp.int32, L)

    def group_body(g, _):
        gbase = g * L
        svx = sx[pl.ds(gbase, L)]
        svy = sy[pl.ds(gbase, L)]
        svz = sz[pl.ds(gbase, L)]
        rvec = jnp.zeros((L,), jnp.float32)
        for u in range(L):
            axv = jnp.full((L,), svx[u], jnp.float32)
            ayv = jnp.full((L,), svy[u], jnp.float32)
            azv = jnp.full((L,), svz[u], jnp.float32)

            def chunk_body(j, acc):
                cbase = j * (L * UNROLL)
                for k in range(UNROLL):
                    off = cbase + k * L
                    dx = bx[pl.ds(off, L)] - axv
                    dy = by[pl.ds(off, L)] - ayv
                    dz = bz[pl.ds(off, L)] - azv
                    d = dx * dx + dy * dy + dz * dz
                    acc = jnp.minimum(acc, d)
                return acc

            acc0 = jnp.full((L,), jnp.inf, jnp.float32)
            acc = lax.fori_loop(0, nchunks, chunk_body, acc0)
            rvec = jnp.where(lane == u, _lane_min_all(acc), rvec)
        outv[pl.ds(gbase, L)] = rvec
        return 0

    lax.fori_loop(0, n_small // L, group_body, 0)


def _sc_chamfer(x1x, x1y, x1z, x2x, x2y, x2z):
    f32 = jnp.float32

    @functools.partial(
        pl.kernel,
        out_type=[jax.ShapeDtypeStruct((B, N), f32),
                  jax.ShapeDtypeStruct((B, SCW), f32)],
        mesh=plsc.VectorSubcoreMesh(core_axis_name="c", subcore_axis_name="s"),
        scratch_types=[
            pltpu.VMEM((RW,), f32), pltpu.VMEM((RW,), f32), pltpu.VMEM((RW,), f32),
            pltpu.VMEM((N,), f32), pltpu.VMEM((N,), f32), pltpu.VMEM((N,), f32),
            pltpu.VMEM((RW,), f32),
        ],
    )
    def sc_kernel(x1x_h, x1y_h, x1z_h, x2x_h, x2y_h, x2z_h, d1_h, d2_h,
                  sxv, syv, szv, bxv, byv, bzv, outv):
        wid = lax.axis_index("s") * 2 + lax.axis_index("c")
        for b in range(B):
            base = wid * RW
            pltpu.sync_copy(x1x_h.at[b, pl.ds(base, RW)], sxv)
            pltpu.sync_copy(x1y_h.at[b, pl.ds(base, RW)], syv)
            pltpu.sync_copy(x1z_h.at[b, pl.ds(base, RW)], szv)
            pltpu.sync_copy(x2x_h.at[b, pl.ds(MT, SCW)], bxv.at[pl.ds(0, SCW)])
            pltpu.sync_copy(x2y_h.at[b, pl.ds(MT, SCW)], byv.at[pl.ds(0, SCW)])
            pltpu.sync_copy(x2z_h.at[b, pl.ds(MT, SCW)], bzv.at[pl.ds(0, SCW)])
            _phase(sxv, syv, szv,
                   bxv.at[pl.ds(0, SCW)], byv.at[pl.ds(0, SCW)],
                   bzv.at[pl.ds(0, SCW)], outv, RW, SCW)
            pltpu.sync_copy(outv, d1_h.at[b, pl.ds(base, RW)])
            @pl.when(wid < NPH2)
            def _phase2():
                cbase = MT + wid * CW
                pltpu.sync_copy(x2x_h.at[b, pl.ds(cbase, CW)], sxv.at[pl.ds(0, CW)])
                pltpu.sync_copy(x2y_h.at[b, pl.ds(cbase, CW)], syv.at[pl.ds(0, CW)])
                pltpu.sync_copy(x2z_h.at[b, pl.ds(cbase, CW)], szv.at[pl.ds(0, CW)])
                pltpu.sync_copy(x1x_h.at[b], bxv)
                pltpu.sync_copy(x1y_h.at[b], byv)
                pltpu.sync_copy(x1z_h.at[b], bzv)
                _phase(sxv.at[pl.ds(0, CW)], syv.at[pl.ds(0, CW)],
                       szv.at[pl.ds(0, CW)], bxv, byv, bzv,
                       outv.at[pl.ds(0, CW)], CW, N)
                pltpu.sync_copy(outv.at[pl.ds(0, CW)],
                                d2_h.at[b, pl.ds(wid * CW, CW)])

    return sc_kernel(x1x, x1y, x1z, x2x, x2y, x2z)


def _tc_body(x1_ref, x2t_ref, d1_ref, d2_ref):
    nb = pl.program_id(1)
    acc = None
    for c in range(C):
        a = x1_ref[0, :, c:c + 1]
        bv = x2t_ref[0, c:c + 1, :]
        diff = a - bv
        sq = diff * diff
        acc = sq if acc is None else acc + sq
    d1_ref[0, 0, :] = jnp.min(acc, axis=1)
    part = jnp.min(acc, axis=0)

    @pl.when(nb == 0)
    def _init():
        d2_ref[0, 0, :] = part

    @pl.when(nb != 0)
    def _accum():
        d2_ref[0, 0, :] = jnp.minimum(d2_ref[0, 0, :], part)


def _tc_chamfer(xyz1, x2t):
    f32 = jnp.float32
    grid = (B, N // BN)
    return pl.pallas_call(
        _tc_body,
        grid=grid,
        in_specs=[
            pl.BlockSpec((1, BN, C), lambda b, nb: (b, nb, 0)),
            pl.BlockSpec((1, C, MT), lambda b, nb: (b, 0, 0)),
        ],
        out_specs=[
            pl.BlockSpec((1, 1, BN), lambda b, nb: (b, 0, nb)),
            pl.BlockSpec((1, 1, MT), lambda b, nb: (b, 0, 0)),
        ],
        out_shape=[
            jax.ShapeDtypeStruct((B, 1, N), f32),
            jax.ShapeDtypeStruct((B, 1, MT), f32),
        ],
    )(xyz1, x2t)


@jax.jit
def kernel(xyz1, xyz2):
    x1t = jnp.transpose(xyz1, (2, 0, 1))
    x2tt = jnp.transpose(xyz2, (2, 0, 1))
    d1_sc, d2_sc = _sc_chamfer(x1t[0], x1t[1], x1t[2],
                               x2tt[0], x2tt[1], x2tt[2])
    x2t = jnp.transpose(xyz2[:, :MT, :], (0, 2, 1))
    d1_tc, d2_tc = _tc_chamfer(xyz1, x2t)
    d1 = jnp.minimum(d1_tc.reshape(B, N), d1_sc)
    d2 = jnp.concatenate([d2_tc.reshape(B, MT), d2_sc], axis=1)
    return d1, d2


# speedup vs baseline: 1.0620x; 1.0620x over previous
import functools

import jax
import jax.numpy as jnp
from jax import lax
from jax.experimental import pallas as pl
from jax.experimental.pallas import tpu as pltpu
from jax.experimental.pallas import tpu_sc as plsc


B, N, M, C = 2, 4096, 4096, 3
SCW = 512
MT = M - SCW
NW = 32
RW = N // NW
CW = 16
NPH2 = SCW // CW
L = 16
UNROLL = 4
BN = 512


_GDN = lax.GatherDimensionNumbers(
    offset_dims=(), collapsed_slice_dims=(0,), start_index_map=(0,))


def _rot(v, s):
    idx = (lax.iota(jnp.int32, L) + s) % L
    return lax.gather(v, idx[:, None], _GDN, slice_sizes=(1,),
                      mode=lax.GatherScatterMode.PROMISE_IN_BOUNDS)


def _lane_min_all(v):
    for s in (8, 4, 2, 1):
        v = jnp.minimum(v, _rot(v, s))
    return v


def _phase(sx, sy, sz, bx, by, bz, outv, n_small, n_big):
    nchunks = n_big // (L * UNROLL)
    lane = lax.iota(jnp.int32, L)

    def group_body(g, _):
        gbase = g * L
        svx = sx[pl.ds(gbase, L)]
        svy = sy[pl.ds(gbase, L)]
        svz = sz[pl.ds(gbase, L)]
        rvec = jnp.zeros((L,), jnp.float32)
        for u in range(L):
            axv = jnp.full((L,), svx[u], jnp.float32)
            ayv = jnp.full((L,), svy[u], jnp.float32)
            azv = jnp.full((L,), svz[u], jnp.float32)

            def chunk_body(j, acc):
                cbase = j * (L * UNROLL)
                for k in range(UNROLL):
                    off = cbase + k * L
                    dx = bx[pl.ds(off, L)] - axv
                    dy = by[pl.ds(off, L)] - ayv
                    dz = bz[pl.ds(off, L)] - azv
                    d = dx * dx + dy * dy + dz * dz
                    acc = jnp.minimum(acc, d)
                return acc

            acc0 = jnp.full((L,), jnp.inf, jnp.float32)
            acc = lax.fori_loop(0, nchunks, chunk_body, acc0)
            rvec = jnp.where(lane == u, _lane_min_all(acc), rvec)
        outv[pl.ds(gbase, L)] = rvec
        return 0

    lax.fori_loop(0, n_small // L, group_body, 0)


def _sc_chamfer(x1x, x1y, x1z, x2x, x2y, x2z):
    f32 = jnp.float32

    @functools.partial(
        pl.kernel,
        out_type=[jax.ShapeDtypeStruct((B, N), f32),
                  jax.ShapeDtypeStruct((B, SCW), f32)],
        mesh=plsc.VectorSubcoreMesh(core_axis_name="c", subcore_axis_name="s"),
        scratch_types=[
            pltpu.VMEM((RW,), f32), pltpu.VMEM((RW,), f32), pltpu.VMEM((RW,), f32),
            pltpu.VMEM((N,), f32), pltpu.VMEM((N,), f32), pltpu.VMEM((N,), f32),
            pltpu.VMEM((RW,), f32),
        ],
    )
    def sc_kernel(x1x_h, x1y_h, x1z_h, x2x_h, x2y_h, x2z_h, d1_h, d2_h,
                  sxv, syv, szv, bxv, byv, bzv, outv):
        wid = lax.axis_index("s") * 2 + lax.axis_index("c")
        for b in range(B):
            base = wid * RW
            pltpu.sync_copy(x1x_h.at[b, pl.ds(base, RW)], sxv)
            pltpu.sync_copy(x1y_h.at[b, pl.ds(base, RW)], syv)
            pltpu.sync_copy(x1z_h.at[b, pl.ds(base, RW)], szv)
            pltpu.sync_copy(x2x_h.at[b, pl.ds(MT, SCW)], bxv.at[pl.ds(0, SCW)])
            pltpu.sync_copy(x2y_h.at[b, pl.ds(MT, SCW)], byv.at[pl.ds(0, SCW)])
            pltpu.sync_copy(x2z_h.at[b, pl.ds(MT, SCW)], bzv.at[pl.ds(0, SCW)])
            _phase(sxv, syv, szv,
                   bxv.at[pl.ds(0, SCW)], byv.at[pl.ds(0, SCW)],
                   bzv.at[pl.ds(0, SCW)], outv, RW, SCW)
            pltpu.sync_copy(outv, d1_h.at[b, pl.ds(base, RW)])
            @pl.when(wid < NPH2)
            def _phase2():
                cbase = MT + wid * CW
                pltpu.sync_copy(x2x_h.at[b, pl.ds(cbase, CW)], sxv.at[pl.ds(0, CW)])
                pltpu.sync_copy(x2y_h.at[b, pl.ds(cbase, CW)], syv.at[pl.ds(0, CW)])
                pltpu.sync_copy(x2z_h.at[b, pl.ds(cbase, CW)], szv.at[pl.ds(0, CW)])
                pltpu.sync_copy(x1x_h.at[b], bxv)
                pltpu.sync_copy(x1y_h.at[b], byv)
                pltpu.sync_copy(x1z_h.at[b], bzv)
                _phase(sxv.at[pl.ds(0, CW)], syv.at[pl.ds(0, CW)],
                       szv.at[pl.ds(0, CW)], bxv, byv, bzv,
                       outv.at[pl.ds(0, CW)], CW, N)
                pltpu.sync_copy(outv.at[pl.ds(0, CW)],
                                d2_h.at[b, pl.ds(wid * CW, CW)])

    return sc_kernel(x1x, x1y, x1z, x2x, x2y, x2z)


def _tc_body(x1_ref, x2t_ref, d1_ref, d2_ref):
    nb = pl.program_id(1)
    acc = None
    for c in range(C):
        a = x1_ref[0, :, c:c + 1]
        bv = x2t_ref[0, c:c + 1, :]
        diff = a - bv
        sq = diff * diff
        acc = sq if acc is None else acc + sq
    d1_ref[0, 0, :] = jnp.min(acc, axis=1)
    part = jnp.min(acc, axis=0)

    @pl.when(nb == 0)
    def _init():
        d2_ref[0, 0, :] = part

    @pl.when(nb != 0)
    def _accum():
        d2_ref[0, 0, :] = jnp.minimum(d2_ref[0, 0, :], part)


def _tc_chamfer(xyz1, x2t):
    f32 = jnp.float32
    grid = (B, N // BN)
    return pl.pallas_call(
        _tc_body,
        grid=grid,
        in_specs=[
            pl.BlockSpec((1, BN, C), lambda b, nb: (b, nb, 0)),
            pl.BlockSpec((1, C, MT), lambda b, nb: (b, 0, 0)),
        ],
        out_specs=[
            pl.BlockSpec((1, 1, BN), lambda b, nb: (b, 0, nb)),
            pl.BlockSpec((1, 1, MT), lambda b, nb: (b, 0, 0)),
        ],
        out_shape=[
            jax.ShapeDtypeStruct((B, 1, N), f32),
            jax.ShapeDtypeStruct((B, 1, MT), f32),
        ],
    )(xyz1, x2t)


@jax.jit
def kernel(xyz1, xyz2):
    x1t = jnp.transpose(xyz1, (2, 0, 1))
    x2tt = jnp.transpose(xyz2, (2, 0, 1))
    d1_sc, d2_sc = _sc_chamfer(x1t[0], x1t[1], x1t[2],
                               x2tt[0], x2tt[1], x2tt[2])
    x2t = jnp.transpose(xyz2[:, :MT, :], (0, 2, 1))
    d1_tc, d2_tc = _tc_chamfer(xyz1, x2t)
    d1 = jnp.minimum(d1_tc.reshape(B, N), d1_sc)
    d2 = jnp.concatenate([d2_tc.reshape(B, MT), d2_sc], axis=1)
    return d1, d2


# speedup vs baseline: 1.0707x; 1.0082x over previous
import functools

import jax
import jax.numpy as jnp
from jax import lax
from jax.experimental import pallas as pl
from jax.experimental.pallas import tpu as pltpu
from jax.experimental.pallas import tpu_sc as plsc


B, N, M, C = 2, 4096, 4096, 3
SCW = 512
MT = M - SCW
NW = 32
RW = N // NW
CW = 16
NPH2 = SCW // CW
L = 16
UNROLL = 4
BN = 2048


_GDN = lax.GatherDimensionNumbers(
    offset_dims=(), collapsed_slice_dims=(0,), start_index_map=(0,))


def _rot(v, s):
    idx = (lax.iota(jnp.int32, L) + s) % L
    return lax.gather(v, idx[:, None], _GDN, slice_sizes=(1,),
                      mode=lax.GatherScatterMode.PROMISE_IN_BOUNDS)


def _lane_min_all(v):
    for s in (8, 4, 2, 1):
        v = jnp.minimum(v, _rot(v, s))
    return v


def _phase(sx, sy, sz, bx, by, bz, outv, n_small, n_big):
    nchunks = n_big // (L * UNROLL)
    lane = lax.iota(jnp.int32, L)

    def group_body(g, _):
        gbase = g * L
        svx = sx[pl.ds(gbase, L)]
        svy = sy[pl.ds(gbase, L)]
        svz = sz[pl.ds(gbase, L)]
        rvec = jnp.zeros((L,), jnp.float32)
        for u in range(L):
            axv = jnp.full((L,), svx[u], jnp.float32)
            ayv = jnp.full((L,), svy[u], jnp.float32)
            azv = jnp.full((L,), svz[u], jnp.float32)

            def chunk_body(j, acc):
                cbase = j * (L * UNROLL)
                for k in range(UNROLL):
                    off = cbase + k * L
                    dx = bx[pl.ds(off, L)] - axv
                    dy = by[pl.ds(off, L)] - ayv
                    dz = bz[pl.ds(off, L)] - azv
                    d = dx * dx + dy * dy + dz * dz
                    acc = jnp.minimum(acc, d)
                return acc

            acc0 = jnp.full((L,), jnp.inf, jnp.float32)
            acc = lax.fori_loop(0, nchunks, chunk_body, acc0)
            rvec = jnp.where(lane == u, _lane_min_all(acc), rvec)
        outv[pl.ds(gbase, L)] = rvec
        return 0

    lax.fori_loop(0, n_small // L, group_body, 0)


def _sc_chamfer(x1x, x1y, x1z, x2x, x2y, x2z):
    f32 = jnp.float32

    @functools.partial(
        pl.kernel,
        out_type=[jax.ShapeDtypeStruct((B, N), f32),
                  jax.ShapeDtypeStruct((B, SCW), f32)],
        mesh=plsc.VectorSubcoreMesh(core_axis_name="c", subcore_axis_name="s"),
        scratch_types=[
            pltpu.VMEM((RW,), f32), pltpu.VMEM((RW,), f32), pltpu.VMEM((RW,), f32),
            pltpu.VMEM((N,), f32), pltpu.VMEM((N,), f32), pltpu.VMEM((N,), f32),
            pltpu.VMEM((RW,), f32),
        ],
    )
    def sc_kernel(x1x_h, x1y_h, x1z_h, x2x_h, x2y_h, x2z_h, d1_h, d2_h,
                  sxv, syv, szv, bxv, byv, bzv, outv):
        wid = lax.axis_index("s") * 2 + lax.axis_index("c")
        for b in range(B):
            base = wid * RW
            pltpu.sync_copy(x1x_h.at[b, pl.ds(base, RW)], sxv)
            pltpu.sync_copy(x1y_h.at[b, pl.ds(base, RW)], syv)
            pltpu.sync_copy(x1z_h.at[b, pl.ds(base, RW)], szv)
            pltpu.sync_copy(x2x_h.at[b, pl.ds(MT, SCW)], bxv.at[pl.ds(0, SCW)])
            pltpu.sync_copy(x2y_h.at[b, pl.ds(MT, SCW)], byv.at[pl.ds(0, SCW)])
            pltpu.sync_copy(x2z_h.at[b, pl.ds(MT, SCW)], bzv.at[pl.ds(0, SCW)])
            _phase(sxv, syv, szv,
                   bxv.at[pl.ds(0, SCW)], byv.at[pl.ds(0, SCW)],
                   bzv.at[pl.ds(0, SCW)], outv, RW, SCW)
            pltpu.sync_copy(outv, d1_h.at[b, pl.ds(base, RW)])
            @pl.when(wid < NPH2)
            def _phase2():
                cbase = MT + wid * CW
                pltpu.sync_copy(x2x_h.at[b, pl.ds(cbase, CW)], sxv.at[pl.ds(0, CW)])
                pltpu.sync_copy(x2y_h.at[b, pl.ds(cbase, CW)], syv.at[pl.ds(0, CW)])
                pltpu.sync_copy(x2z_h.at[b, pl.ds(cbase, CW)], szv.at[pl.ds(0, CW)])
                pltpu.sync_copy(x1x_h.at[b], bxv)
                pltpu.sync_copy(x1y_h.at[b], byv)
                pltpu.sync_copy(x1z_h.at[b], bzv)
                _phase(sxv.at[pl.ds(0, CW)], syv.at[pl.ds(0, CW)],
                       szv.at[pl.ds(0, CW)], bxv, byv, bzv,
                       outv.at[pl.ds(0, CW)], CW, N)
                pltpu.sync_copy(outv.at[pl.ds(0, CW)],
                                d2_h.at[b, pl.ds(wid * CW, CW)])

    return sc_kernel(x1x, x1y, x1z, x2x, x2y, x2z)


def _tc_body(x1_ref, x2t_ref, d1_ref, d2_ref):
    nb = pl.program_id(1)
    acc = None
    for c in range(C):
        a = x1_ref[0, :, c:c + 1]
        bv = x2t_ref[0, c:c + 1, :]
        diff = a - bv
        sq = diff * diff
        acc = sq if acc is None else acc + sq
    d1_ref[0, 0, :] = jnp.min(acc, axis=1)
    part = jnp.min(acc, axis=0)

    @pl.when(nb == 0)
    def _init():
        d2_ref[0, 0, :] = part

    @pl.when(nb != 0)
    def _accum():
        d2_ref[0, 0, :] = jnp.minimum(d2_ref[0, 0, :], part)


def _tc_chamfer(xyz1, x2t):
    f32 = jnp.float32
    grid = (B, N // BN)
    return pl.pallas_call(
        _tc_body,
        grid=grid,
        in_specs=[
            pl.BlockSpec((1, BN, C), lambda b, nb: (b, nb, 0)),
            pl.BlockSpec((1, C, MT), lambda b, nb: (b, 0, 0)),
        ],
        out_specs=[
            pl.BlockSpec((1, 1, BN), lambda b, nb: (b, 0, nb)),
            pl.BlockSpec((1, 1, MT), lambda b, nb: (b, 0, 0)),
        ],
        out_shape=[
            jax.ShapeDtypeStruct((B, 1, N), f32),
            jax.ShapeDtypeStruct((B, 1, MT), f32),
        ],
    )(xyz1, x2t)


@jax.jit
def kernel(xyz1, xyz2):
    x1t = jnp.transpose(xyz1, (2, 0, 1))
    x2tt = jnp.transpose(xyz2, (2, 0, 1))
    d1_sc, d2_sc = _sc_chamfer(x1t[0], x1t[1], x1t[2],
                               x2tt[0], x2tt[1], x2tt[2])
    x2t = jnp.transpose(xyz2[:, :MT, :], (0, 2, 1))
    d1_tc, d2_tc = _tc_chamfer(xyz1, x2t)
    d1 = jnp.minimum(d1_tc.reshape(B, N), d1_sc)
    d2 = jnp.concatenate([d2_tc.reshape(B, MT), d2_sc], axis=1)
    return d1, d2
